# baseline (speedup 1.0000x reference)
import jax
import jax.numpy as jnp
from jax.experimental import pallas as pl

_ROWS_PER_BLOCK = 2048


def _rowdot_kernel(gu_ref, gi_ref, out_ref):
    out_ref[:] = jnp.sum(gu_ref[:] * gi_ref[:], axis=1, keepdims=True)


def kernel(gu, gi):
    n, k = gu.shape
    grid = (n // _ROWS_PER_BLOCK,)
    out = pl.pallas_call(
        _rowdot_kernel,
        grid=grid,
        in_specs=[
            pl.BlockSpec((_ROWS_PER_BLOCK, k), lambda i: (i, 0)),
            pl.BlockSpec((_ROWS_PER_BLOCK, k), lambda i: (i, 0)),
        ],
        out_specs=pl.BlockSpec((_ROWS_PER_BLOCK, 1), lambda i: (i, 0)),
        out_shape=jax.ShapeDtypeStruct((n, 1), jnp.float32),
    )(gu, gi)
    return out.reshape(n)


# speedup vs baseline: 1.2001x; 1.2001x over previous
import jax
import jax.numpy as jnp
from jax.experimental import pallas as pl

_ROWS_PER_BLOCK = 2048


def _rowdot_kernel(gu_ref, gi_ref, out_ref):
    out_ref[:] = jnp.sum(gu_ref[:] * gi_ref[:], axis=1)


def kernel(gu, gi):
    n, k = gu.shape
    grid = (n // _ROWS_PER_BLOCK,)
    out = pl.pallas_call(
        _rowdot_kernel,
        grid=grid,
        in_specs=[
            pl.BlockSpec((_ROWS_PER_BLOCK, k), lambda i: (i, 0)),
            pl.BlockSpec((_ROWS_PER_BLOCK, k), lambda i: (i, 0)),
        ],
        out_specs=pl.BlockSpec((_ROWS_PER_BLOCK,), lambda i: (i,)),
        out_shape=jax.ShapeDtypeStruct((n,), jnp.float32),
    )(gu, gi)
    return out
